# initial kernel scaffold (unmeasured)
import jax
import jax.numpy as jnp
from jax import lax
from jax.experimental import pallas as pl
from jax.experimental.pallas import tpu as pltpu

N_DEV = 4


def kernel(x, w_mat):
    m_total, k_per = x.shape
    k_per2, n = w_mat.shape
    assert k_per == k_per2
    m_per = m_total // N_DEV
    half = n // 2

    def body(x_ref, w_ref, out_ref,
             p_cw, p_ccw, recv_cw, recv_ccw,
             cw_send_sems, cw_recv_sems, ccw_send_sems, ccw_recv_sems):
        d = lax.axis_index("i")
        left = lax.rem(d + N_DEV - 1, N_DEV)
        right = lax.rem(d + 1, N_DEV)

        barrier_sem = pltpu.get_barrier_semaphore()
        for nbr in (left, right):
            pl.semaphore_signal(
                barrier_sem, inc=1,
                device_id=(nbr,), device_id_type=pl.DeviceIdType.MESH,
            )
        pl.semaphore_wait(barrier_sem, 2)

        for j in range(N_DEV):
            c = lax.rem(d + 2 * N_DEV - 1 - j, N_DEV)
            part = jnp.dot(
                x_ref[pl.ds(c * m_per, m_per), :], w_ref[...],
                preferred_element_type=jnp.float32,
            )
            p_cw[j, :, :] = part[:, :half]
            jccw = (2 - j) if j < 3 else 3
            p_ccw[jccw, :, :] = part[:, half:]

        for s in range(N_DEV - 1):
            cw = pltpu.make_async_remote_copy(
                src_ref=p_cw.at[s],
                dst_ref=recv_cw.at[s],
                send_sem=cw_send_sems.at[s],
                recv_sem=cw_recv_sems.at[s],
                device_id=(right,),
                device_id_type=pl.DeviceIdType.MESH,
            )
            ccw = pltpu.make_async_remote_copy(
                src_ref=p_ccw.at[s],
                dst_ref=recv_ccw.at[s],
                send_sem=ccw_send_sems.at[s],
                recv_sem=ccw_recv_sems.at[s],
                device_id=(left,),
                device_id_type=pl.DeviceIdType.MESH,
            )
            cw.start()
            ccw.start()
            cw.wait()
            ccw.wait()
            p_cw[s + 1, :, :] = p_cw[s + 1, :, :] + recv_cw[s, :, :]
            p_ccw[s + 1, :, :] = p_ccw[s + 1, :, :] + recv_ccw[s, :, :]

        out_ref[:, :half] = jnp.maximum(p_cw[3, :, :], 0.0)
        out_ref[:, half:] = jnp.maximum(p_ccw[3, :, :], 0.0)

    return pl.pallas_call(
        body,
        out_shape=jax.ShapeDtypeStruct((m_per, n), jnp.float32),
        in_specs=[
            pl.BlockSpec(memory_space=pltpu.VMEM),
            pl.BlockSpec(memory_space=pltpu.VMEM),
        ],
        out_specs=pl.BlockSpec(memory_space=pltpu.VMEM),
        scratch_shapes=[
            pltpu.VMEM((N_DEV, m_per, half), jnp.float32),
            pltpu.VMEM((N_DEV, m_per, half), jnp.float32),
            pltpu.VMEM((N_DEV - 1, m_per, half), jnp.float32),
            pltpu.VMEM((N_DEV - 1, m_per, half), jnp.float32),
            pltpu.SemaphoreType.DMA((N_DEV - 1,)),
            pltpu.SemaphoreType.DMA((N_DEV - 1,)),
            pltpu.SemaphoreType.DMA((N_DEV - 1,)),
            pltpu.SemaphoreType.DMA((N_DEV - 1,)),
        ],
        compiler_params=pltpu.CompilerParams(collective_id=0),
    )(x, w_mat)


# baseline (device time: 181026 ns/iter reference)
import jax
import jax.numpy as jnp
from jax import lax
from jax.experimental import pallas as pl
from jax.experimental.pallas import tpu as pltpu

N_DEV = 4


def kernel(x, w_mat):
    m_total, k_per = x.shape
    k_per2, n = w_mat.shape
    assert k_per == k_per2
    m_per = m_total // N_DEV
    half = n // 2

    def body(x_ref, w_ref, out_ref,
             p0, recv_cw, recv_ccw,
             cw_send_sems, cw_recv_sems, ccw_send_sems, ccw_recv_sems,
             credit_cw, credit_ccw):
        d = lax.axis_index("i")
        left = lax.rem(d + N_DEV - 1, N_DEV)
        right = lax.rem(d + 1, N_DEV)

        def xdot(chunk_off, w_cols):
            c = lax.rem(d + chunk_off, N_DEV)
            return jnp.dot(
                x_ref[pl.ds(c * m_per, m_per), :], w_ref[:, w_cols],
                preferred_element_type=jnp.float32,
            )

        barrier_sem = pltpu.get_barrier_semaphore()
        for nbr in (left, right):
            pl.semaphore_signal(
                barrier_sem, inc=1,
                device_id=(nbr,), device_id_type=pl.DeviceIdType.MESH,
            )
        pl.semaphore_wait(barrier_sem, 2)

        p0[0, :, :] = xdot(N_DEV - 1, pl.ds(0, half))
        p0[1, :, :] = xdot(1, pl.ds(half, half))

        def ring_step(s, cw_src, cw_dst, ccw_src, ccw_dst):
            cw = pltpu.make_async_remote_copy(
                src_ref=cw_src, dst_ref=cw_dst,
                send_sem=cw_send_sems.at[s], recv_sem=cw_recv_sems.at[s],
                device_id=(right,), device_id_type=pl.DeviceIdType.MESH,
            )
            ccw = pltpu.make_async_remote_copy(
                src_ref=ccw_src, dst_ref=ccw_dst,
                send_sem=ccw_send_sems.at[s], recv_sem=ccw_recv_sems.at[s],
                device_id=(left,), device_id_type=pl.DeviceIdType.MESH,
            )
            cw.start()
            ccw.start()
            cw.wait()
            ccw.wait()

        ring_step(0, p0.at[0], recv_cw.at[0], p0.at[1], recv_ccw.at[0])

        recv_cw[0, :, :] = recv_cw[0, :, :] + xdot(2, pl.ds(0, half))
        recv_ccw[0, :, :] = recv_ccw[0, :, :] + xdot(2, pl.ds(half, half))
        ring_step(1, recv_cw.at[0], recv_cw.at[1], recv_ccw.at[0], recv_ccw.at[1])

        pl.semaphore_signal(
            credit_cw, inc=1,
            device_id=(left,), device_id_type=pl.DeviceIdType.MESH,
        )
        pl.semaphore_signal(
            credit_ccw, inc=1,
            device_id=(right,), device_id_type=pl.DeviceIdType.MESH,
        )

        recv_cw[1, :, :] = recv_cw[1, :, :] + xdot(1, pl.ds(0, half))
        recv_ccw[1, :, :] = recv_ccw[1, :, :] + xdot(N_DEV - 1, pl.ds(half, half))
        pl.semaphore_wait(credit_cw, 1)
        pl.semaphore_wait(credit_ccw, 1)
        ring_step(2, recv_cw.at[1], recv_cw.at[0], recv_ccw.at[1], recv_ccw.at[0])

        out_ref[:, :half] = jnp.maximum(
            recv_cw[0, :, :] + xdot(0, pl.ds(0, half)), 0.0
        )
        out_ref[:, half:] = jnp.maximum(
            recv_ccw[0, :, :] + xdot(0, pl.ds(half, half)), 0.0
        )

    return pl.pallas_call(
        body,
        out_shape=jax.ShapeDtypeStruct((m_per, n), jnp.float32),
        in_specs=[
            pl.BlockSpec(memory_space=pltpu.VMEM),
            pl.BlockSpec(memory_space=pltpu.VMEM),
        ],
        out_specs=pl.BlockSpec(memory_space=pltpu.VMEM),
        scratch_shapes=[
            pltpu.VMEM((2, m_per, half), jnp.float32),
            pltpu.VMEM((2, m_per, half), jnp.float32),
            pltpu.VMEM((2, m_per, half), jnp.float32),
            pltpu.SemaphoreType.DMA((N_DEV - 1,)),
            pltpu.SemaphoreType.DMA((N_DEV - 1,)),
            pltpu.SemaphoreType.DMA((N_DEV - 1,)),
            pltpu.SemaphoreType.DMA((N_DEV - 1,)),
            pltpu.SemaphoreType.REGULAR,
            pltpu.SemaphoreType.REGULAR,
        ],
        compiler_params=pltpu.CompilerParams(
            collective_id=0,
            vmem_limit_bytes=128 * 1024 * 1024,
        ),
    )(x, w_mat)


# device time: 169035 ns/iter; 1.0709x vs baseline; 1.0709x over previous
import jax
import jax.numpy as jnp
from jax import lax
from jax.experimental import pallas as pl
from jax.experimental.pallas import tpu as pltpu

N_DEV = 4


def kernel(x, w_mat):
    m_total, k_per = x.shape
    k_per2, n = w_mat.shape
    assert k_per == k_per2
    m_per = m_total // N_DEV
    half = n // 2

    def body(x_ref, w_ref, out_ref,
             p0, recv_cw, recv_ccw,
             cw_send_sems, cw_recv_sems, ccw_send_sems, ccw_recv_sems,
             credit_cw, credit_ccw):
        d = lax.axis_index("i")
        left = lax.rem(d + N_DEV - 1, N_DEV)
        right = lax.rem(d + 1, N_DEV)

        def xdot(chunk_off, w_cols):
            c = lax.rem(d + chunk_off, N_DEV)
            return jnp.dot(
                x_ref[pl.ds(c * m_per, m_per), :], w_ref[:, w_cols],
                preferred_element_type=jnp.float32,
            )

        barrier_sem = pltpu.get_barrier_semaphore()
        for nbr in (left, right):
            pl.semaphore_signal(
                barrier_sem, inc=1,
                device_id=(nbr,), device_id_type=pl.DeviceIdType.MESH,
            )
        pl.semaphore_wait(barrier_sem, 2)

        p0[0, :, :] = xdot(N_DEV - 1, pl.ds(0, half))
        p0[1, :, :] = xdot(1, pl.ds(half, half))

        def ring_rdmas(s, cw_src, cw_dst, ccw_src, ccw_dst):
            cw = pltpu.make_async_remote_copy(
                src_ref=cw_src, dst_ref=cw_dst,
                send_sem=cw_send_sems.at[s], recv_sem=cw_recv_sems.at[s],
                device_id=(right,), device_id_type=pl.DeviceIdType.MESH,
            )
            ccw = pltpu.make_async_remote_copy(
                src_ref=ccw_src, dst_ref=ccw_dst,
                send_sem=ccw_send_sems.at[s], recv_sem=ccw_recv_sems.at[s],
                device_id=(left,), device_id_type=pl.DeviceIdType.MESH,
            )
            cw.start()
            ccw.start()
            return cw, ccw


        cw, ccw = ring_rdmas(0, p0.at[0], recv_cw.at[0], p0.at[1], recv_ccw.at[0])
        out_ref[:, :half] = xdot(2, pl.ds(0, half))
        out_ref[:, half:] = xdot(2, pl.ds(half, half))
        cw.wait()
        ccw.wait()
        recv_cw[0, :, :] = recv_cw[0, :, :] + out_ref[:, :half]
        recv_ccw[0, :, :] = recv_ccw[0, :, :] + out_ref[:, half:]

        cw, ccw = ring_rdmas(1, recv_cw.at[0], recv_cw.at[1],
                             recv_ccw.at[0], recv_ccw.at[1])
        out_ref[:, :half] = xdot(1, pl.ds(0, half))
        out_ref[:, half:] = xdot(N_DEV - 1, pl.ds(half, half))
        cw.wait()
        ccw.wait()
        recv_cw[1, :, :] = recv_cw[1, :, :] + out_ref[:, :half]
        recv_ccw[1, :, :] = recv_ccw[1, :, :] + out_ref[:, half:]

        pl.semaphore_signal(
            credit_cw, inc=1,
            device_id=(left,), device_id_type=pl.DeviceIdType.MESH,
        )
        pl.semaphore_signal(
            credit_ccw, inc=1,
            device_id=(right,), device_id_type=pl.DeviceIdType.MESH,
        )
        pl.semaphore_wait(credit_cw, 1)
        pl.semaphore_wait(credit_ccw, 1)

        cw, ccw = ring_rdmas(2, recv_cw.at[1], recv_cw.at[0],
                             recv_ccw.at[1], recv_ccw.at[0])
        out_ref[:, :half] = xdot(0, pl.ds(0, half))
        out_ref[:, half:] = xdot(0, pl.ds(half, half))
        cw.wait()
        ccw.wait()

        out_ref[:, :half] = jnp.maximum(
            recv_cw[0, :, :] + out_ref[:, :half], 0.0
        )
        out_ref[:, half:] = jnp.maximum(
            recv_ccw[0, :, :] + out_ref[:, half:], 0.0
        )

    return pl.pallas_call(
        body,
        out_shape=jax.ShapeDtypeStruct((m_per, n), jnp.float32),
        in_specs=[
            pl.BlockSpec(memory_space=pltpu.VMEM),
            pl.BlockSpec(memory_space=pltpu.VMEM),
        ],
        out_specs=pl.BlockSpec(memory_space=pltpu.VMEM),
        scratch_shapes=[
            pltpu.VMEM((2, m_per, half), jnp.float32),
            pltpu.VMEM((2, m_per, half), jnp.float32),
            pltpu.VMEM((2, m_per, half), jnp.float32),
            pltpu.SemaphoreType.DMA((N_DEV - 1,)),
            pltpu.SemaphoreType.DMA((N_DEV - 1,)),
            pltpu.SemaphoreType.DMA((N_DEV - 1,)),
            pltpu.SemaphoreType.DMA((N_DEV - 1,)),
            pltpu.SemaphoreType.REGULAR,
            pltpu.SemaphoreType.REGULAR,
        ],
        compiler_params=pltpu.CompilerParams(
            collective_id=0,
            vmem_limit_bytes=128 * 1024 * 1024,
        ),
    )(x, w_mat)


# device time: 162032 ns/iter; 1.1172x vs baseline; 1.0432x over previous
import jax
import jax.numpy as jnp
from jax import lax
from jax.experimental import pallas as pl
from jax.experimental.pallas import tpu as pltpu

N_DEV = 4
R = 2


def kernel(x, w_mat):
    m_total, k_per = x.shape
    k_per2, n = w_mat.shape
    assert k_per == k_per2
    m_per = m_total // N_DEV
    half = n // 2
    sub = m_per // R

    def body(x_ref, w_ref, out_ref,
             p0, recv_cw, recv_ccw,
             cw_send_sems, cw_recv_sems, ccw_send_sems, ccw_recv_sems,
             credit_cw, credit_ccw):
        d = lax.axis_index("i")
        left = lax.rem(d + N_DEV - 1, N_DEV)
        right = lax.rem(d + 1, N_DEV)

        lcols = pl.ds(0, half)
        rcols = pl.ds(half, half)

        def rows(r):
            return pl.ds(r * sub, sub)

        def xdot(chunk_off, w_cols, r=None):
            c = lax.rem(d + chunk_off, N_DEV)
            off = c * m_per if r is None else c * m_per + r * sub
            return jnp.dot(
                x_ref[pl.ds(off, m_per if r is None else sub), :],
                w_ref[:, w_cols],
                preferred_element_type=jnp.float32,
            )

        def mk(s, r, src, dst, send_sems, recv_sems, tgt):
            return pltpu.make_async_remote_copy(
                src_ref=src, dst_ref=dst,
                send_sem=send_sems.at[s * R + r],
                recv_sem=recv_sems.at[s * R + r],
                device_id=(tgt,), device_id_type=pl.DeviceIdType.MESH,
            )

        def mk_cw(s, r, src, dst):
            return mk(s, r, src, dst, cw_send_sems, cw_recv_sems, right)

        def mk_ccw(s, r, src, dst):
            return mk(s, r, src, dst, ccw_send_sems, ccw_recv_sems, left)

        barrier_sem = pltpu.get_barrier_semaphore()
        for nbr in (left, right):
            pl.semaphore_signal(
                barrier_sem, inc=1,
                device_id=(nbr,), device_id_type=pl.DeviceIdType.MESH,
            )
        pl.semaphore_wait(barrier_sem, 2)

        cw0, ccw0 = [], []
        for r in range(R):
            p0[0, rows(r), :] = xdot(N_DEV - 1, lcols, r)
            cw0.append(mk_cw(0, r, p0.at[0, rows(r)], recv_cw.at[0, rows(r)]))
            cw0[r].start()
            p0[1, rows(r), :] = xdot(1, rcols, r)
            ccw0.append(mk_ccw(0, r, p0.at[1, rows(r)], recv_ccw.at[0, rows(r)]))
            ccw0[r].start()

        out_ref[:, lcols] = xdot(2, lcols)
        out_ref[:, rcols] = xdot(2, rcols)

        cw1, ccw1 = [], []
        for r in range(R):
            cw0[r].wait_recv()
            recv_cw[0, rows(r), :] = (
                recv_cw[0, rows(r), :] + out_ref[rows(r), lcols]
            )
            cw1.append(mk_cw(1, r, recv_cw.at[0, rows(r)], recv_cw.at[1, rows(r)]))
            cw1[r].start()
            ccw0[r].wait_recv()
            recv_ccw[0, rows(r), :] = (
                recv_ccw[0, rows(r), :] + out_ref[rows(r), rcols]
            )
            ccw1.append(mk_ccw(1, r, recv_ccw.at[0, rows(r)], recv_ccw.at[1, rows(r)]))
            ccw1[r].start()
        for r in range(R):
            cw0[r].wait_send()
            ccw0[r].wait_send()

        out_ref[:, lcols] = xdot(1, lcols)
        out_ref[:, rcols] = xdot(N_DEV - 1, rcols)

        for r in range(R):
            cw1[r].wait_send()
            pl.semaphore_signal(
                credit_cw, inc=1,
                device_id=(left,), device_id_type=pl.DeviceIdType.MESH,
            )
            ccw1[r].wait_send()
            pl.semaphore_signal(
                credit_ccw, inc=1,
                device_id=(right,), device_id_type=pl.DeviceIdType.MESH,
            )

        cw2, ccw2 = [], []
        for r in range(R):
            cw1[r].wait_recv()
            recv_cw[1, rows(r), :] = (
                recv_cw[1, rows(r), :] + out_ref[rows(r), lcols]
            )
            pl.semaphore_wait(credit_cw, 1)
            cw2.append(mk_cw(2, r, recv_cw.at[1, rows(r)], recv_cw.at[0, rows(r)]))
            cw2[r].start()
            ccw1[r].wait_recv()
            recv_ccw[1, rows(r), :] = (
                recv_ccw[1, rows(r), :] + out_ref[rows(r), rcols]
            )
            pl.semaphore_wait(credit_ccw, 1)
            ccw2.append(mk_ccw(2, r, recv_ccw.at[1, rows(r)], recv_ccw.at[0, rows(r)]))
            ccw2[r].start()

        out_ref[:, lcols] = xdot(0, lcols)
        out_ref[:, rcols] = xdot(0, rcols)

        for r in range(R):
            cw2[r].wait_recv()
            out_ref[rows(r), lcols] = jnp.maximum(
                recv_cw[0, rows(r), :] + out_ref[rows(r), lcols], 0.0
            )
            ccw2[r].wait_recv()
            out_ref[rows(r), rcols] = jnp.maximum(
                recv_ccw[0, rows(r), :] + out_ref[rows(r), rcols], 0.0
            )
        for r in range(R):
            cw2[r].wait_send()
            ccw2[r].wait_send()

    return pl.pallas_call(
        body,
        out_shape=jax.ShapeDtypeStruct((m_per, n), jnp.float32),
        in_specs=[
            pl.BlockSpec(memory_space=pltpu.VMEM),
            pl.BlockSpec(memory_space=pltpu.VMEM),
        ],
        out_specs=pl.BlockSpec(memory_space=pltpu.VMEM),
        scratch_shapes=[
            pltpu.VMEM((2, m_per, half), jnp.float32),
            pltpu.VMEM((2, m_per, half), jnp.float32),
            pltpu.VMEM((2, m_per, half), jnp.float32),
            pltpu.SemaphoreType.DMA(((N_DEV - 1) * R,)),
            pltpu.SemaphoreType.DMA(((N_DEV - 1) * R,)),
            pltpu.SemaphoreType.DMA(((N_DEV - 1) * R,)),
            pltpu.SemaphoreType.DMA(((N_DEV - 1) * R,)),
            pltpu.SemaphoreType.REGULAR,
            pltpu.SemaphoreType.REGULAR,
        ],
        compiler_params=pltpu.CompilerParams(
            collective_id=0,
            vmem_limit_bytes=128 * 1024 * 1024,
        ),
    )(x, w_mat)


# device time: 160702 ns/iter; 1.1265x vs baseline; 1.0083x over previous
import jax
import jax.numpy as jnp
from jax import lax
from jax.experimental import pallas as pl
from jax.experimental.pallas import tpu as pltpu

N_DEV = 4
R = 4


def kernel(x, w_mat):
    m_total, k_per = x.shape
    k_per2, n = w_mat.shape
    assert k_per == k_per2
    m_per = m_total // N_DEV
    half = n // 2
    sub = m_per // R

    def body(x_ref, w_ref, out_ref,
             p0, recv_cw, recv_ccw,
             cw_send_sems, cw_recv_sems, ccw_send_sems, ccw_recv_sems,
             credit_cw, credit_ccw):
        d = lax.axis_index("i")
        left = lax.rem(d + N_DEV - 1, N_DEV)
        right = lax.rem(d + 1, N_DEV)

        lcols = pl.ds(0, half)
        rcols = pl.ds(half, half)

        def rows(r):
            return pl.ds(r * sub, sub)

        def xdot(chunk_off, w_cols, r=None):
            c = lax.rem(d + chunk_off, N_DEV)
            off = c * m_per if r is None else c * m_per + r * sub
            return jnp.dot(
                x_ref[pl.ds(off, m_per if r is None else sub), :],
                w_ref[:, w_cols],
                preferred_element_type=jnp.float32,
            )

        def mk(s, r, src, dst, send_sems, recv_sems, tgt):
            return pltpu.make_async_remote_copy(
                src_ref=src, dst_ref=dst,
                send_sem=send_sems.at[s * R + r],
                recv_sem=recv_sems.at[s * R + r],
                device_id=(tgt,), device_id_type=pl.DeviceIdType.MESH,
            )

        def mk_cw(s, r, src, dst):
            return mk(s, r, src, dst, cw_send_sems, cw_recv_sems, right)

        def mk_ccw(s, r, src, dst):
            return mk(s, r, src, dst, ccw_send_sems, ccw_recv_sems, left)

        barrier_sem = pltpu.get_barrier_semaphore()
        for nbr in (left, right):
            pl.semaphore_signal(
                barrier_sem, inc=1,
                device_id=(nbr,), device_id_type=pl.DeviceIdType.MESH,
            )
        pl.semaphore_wait(barrier_sem, 2)

        cw0, ccw0 = [], []
        for r in range(R):
            p0[0, rows(r), :] = xdot(N_DEV - 1, lcols, r)
            cw0.append(mk_cw(0, r, p0.at[0, rows(r)], recv_cw.at[0, rows(r)]))
            cw0[r].start()
            p0[1, rows(r), :] = xdot(1, rcols, r)
            ccw0.append(mk_ccw(0, r, p0.at[1, rows(r)], recv_ccw.at[0, rows(r)]))
            ccw0[r].start()

        out_ref[:, lcols] = xdot(2, lcols)
        out_ref[:, rcols] = xdot(2, rcols)

        cw1, ccw1 = [], []
        for r in range(R):
            cw0[r].wait_recv()
            recv_cw[0, rows(r), :] = (
                recv_cw[0, rows(r), :] + out_ref[rows(r), lcols]
            )
            cw1.append(mk_cw(1, r, recv_cw.at[0, rows(r)], recv_cw.at[1, rows(r)]))
            cw1[r].start()
            ccw0[r].wait_recv()
            recv_ccw[0, rows(r), :] = (
                recv_ccw[0, rows(r), :] + out_ref[rows(r), rcols]
            )
            ccw1.append(mk_ccw(1, r, recv_ccw.at[0, rows(r)], recv_ccw.at[1, rows(r)]))
            ccw1[r].start()
        for r in range(R):
            cw0[r].wait_send()
            ccw0[r].wait_send()

        out_ref[:, lcols] = xdot(1, lcols)
        out_ref[:, rcols] = xdot(N_DEV - 1, rcols)

        for r in range(R):
            cw1[r].wait_send()
            pl.semaphore_signal(
                credit_cw, inc=1,
                device_id=(left,), device_id_type=pl.DeviceIdType.MESH,
            )
            ccw1[r].wait_send()
            pl.semaphore_signal(
                credit_ccw, inc=1,
                device_id=(right,), device_id_type=pl.DeviceIdType.MESH,
            )

        cw2, ccw2 = [], []
        for r in range(R):
            cw1[r].wait_recv()
            recv_cw[1, rows(r), :] = (
                recv_cw[1, rows(r), :] + out_ref[rows(r), lcols]
            )
            pl.semaphore_wait(credit_cw, 1)
            cw2.append(mk_cw(2, r, recv_cw.at[1, rows(r)], recv_cw.at[0, rows(r)]))
            cw2[r].start()
            ccw1[r].wait_recv()
            recv_ccw[1, rows(r), :] = (
                recv_ccw[1, rows(r), :] + out_ref[rows(r), rcols]
            )
            pl.semaphore_wait(credit_ccw, 1)
            ccw2.append(mk_ccw(2, r, recv_ccw.at[1, rows(r)], recv_ccw.at[0, rows(r)]))
            ccw2[r].start()

        out_ref[:, lcols] = xdot(0, lcols)
        out_ref[:, rcols] = xdot(0, rcols)

        for r in range(R):
            cw2[r].wait_recv()
            out_ref[rows(r), lcols] = jnp.maximum(
                recv_cw[0, rows(r), :] + out_ref[rows(r), lcols], 0.0
            )
            ccw2[r].wait_recv()
            out_ref[rows(r), rcols] = jnp.maximum(
                recv_ccw[0, rows(r), :] + out_ref[rows(r), rcols], 0.0
            )
        for r in range(R):
            cw2[r].wait_send()
            ccw2[r].wait_send()

    return pl.pallas_call(
        body,
        out_shape=jax.ShapeDtypeStruct((m_per, n), jnp.float32),
        in_specs=[
            pl.BlockSpec(memory_space=pltpu.VMEM),
            pl.BlockSpec(memory_space=pltpu.VMEM),
        ],
        out_specs=pl.BlockSpec(memory_space=pltpu.VMEM),
        scratch_shapes=[
            pltpu.VMEM((2, m_per, half), jnp.float32),
            pltpu.VMEM((2, m_per, half), jnp.float32),
            pltpu.VMEM((2, m_per, half), jnp.float32),
            pltpu.SemaphoreType.DMA(((N_DEV - 1) * R,)),
            pltpu.SemaphoreType.DMA(((N_DEV - 1) * R,)),
            pltpu.SemaphoreType.DMA(((N_DEV - 1) * R,)),
            pltpu.SemaphoreType.DMA(((N_DEV - 1) * R,)),
            pltpu.SemaphoreType.REGULAR,
            pltpu.SemaphoreType.REGULAR,
        ],
        compiler_params=pltpu.CompilerParams(
            collective_id=0,
            vmem_limit_bytes=128 * 1024 * 1024,
        ),
    )(x, w_mat)


# device time: 156621 ns/iter; 1.1558x vs baseline; 1.0261x over previous
import jax
import jax.numpy as jnp
from jax import lax
from jax.experimental import pallas as pl
from jax.experimental.pallas import tpu as pltpu

N_DEV = 4
R = 4


def kernel(x, w_mat):
    m_total, k_per = x.shape
    k_per2, n = w_mat.shape
    assert k_per == k_per2
    m_per = m_total // N_DEV
    half = n // 2
    sub = m_per // R

    def body(x_hbm, w_hbm, out_ref,
             p0, recv_cw, recv_ccw, xc, wv,
             cw_send_sems, cw_recv_sems, ccw_send_sems, ccw_recv_sems,
             load_sems, credit_cw, credit_ccw):
        d = lax.axis_index("i")
        left = lax.rem(d + N_DEV - 1, N_DEV)
        right = lax.rem(d + 1, N_DEV)

        lcols = pl.ds(0, half)
        rcols = pl.ds(half, half)

        def rows(r):
            return pl.ds(r * sub, sub)

        XOFF = (N_DEV - 1, 1, 2, 0)
        loads = []
        for j, off in enumerate(XOFF):
            c = lax.rem(d + off, N_DEV)
            loads.append(pltpu.make_async_copy(
                x_hbm.at[pl.ds(c * m_per, m_per), :], xc.at[j],
                load_sems.at[j],
            ))
        wload = pltpu.make_async_copy(w_hbm, wv, load_sems.at[N_DEV])
        loads[0].start()
        wload.start()
        loads[1].start()
        loads[2].start()
        loads[3].start()

        def xdot(j, w_cols, r=None):
            off = 0 if r is None else r * sub
            return jnp.dot(
                xc[j, pl.ds(off, m_per if r is None else sub), :],
                wv[:, w_cols],
                preferred_element_type=jnp.float32,
            )

        def mk(s, r, src, dst, send_sems, recv_sems, tgt):
            return pltpu.make_async_remote_copy(
                src_ref=src, dst_ref=dst,
                send_sem=send_sems.at[s * R + r],
                recv_sem=recv_sems.at[s * R + r],
                device_id=(tgt,), device_id_type=pl.DeviceIdType.MESH,
            )

        def mk_cw(s, r, src, dst):
            return mk(s, r, src, dst, cw_send_sems, cw_recv_sems, right)

        def mk_ccw(s, r, src, dst):
            return mk(s, r, src, dst, ccw_send_sems, ccw_recv_sems, left)

        barrier_sem = pltpu.get_barrier_semaphore()
        for nbr in (left, right):
            pl.semaphore_signal(
                barrier_sem, inc=1,
                device_id=(nbr,), device_id_type=pl.DeviceIdType.MESH,
            )
        pl.semaphore_wait(barrier_sem, 2)

        loads[0].wait()
        wload.wait()
        cw0, ccw0 = [], []
        for r in range(R):
            p0[0, rows(r), :] = xdot(0, lcols, r)
            cw0.append(mk_cw(0, r, p0.at[0, rows(r)], recv_cw.at[0, rows(r)]))
            cw0[r].start()
            if r == 0:
                loads[1].wait()
            p0[1, rows(r), :] = xdot(1, rcols, r)
            ccw0.append(mk_ccw(0, r, p0.at[1, rows(r)], recv_ccw.at[0, rows(r)]))
            ccw0[r].start()

        loads[2].wait()
        out_ref[:, lcols] = xdot(2, lcols)
        out_ref[:, rcols] = xdot(2, rcols)

        cw1, ccw1 = [], []
        for r in range(R):
            cw0[r].wait_recv()
            recv_cw[0, rows(r), :] = (
                recv_cw[0, rows(r), :] + out_ref[rows(r), lcols]
            )
            cw1.append(mk_cw(1, r, recv_cw.at[0, rows(r)], recv_cw.at[1, rows(r)]))
            cw1[r].start()
            ccw0[r].wait_recv()
            recv_ccw[0, rows(r), :] = (
                recv_ccw[0, rows(r), :] + out_ref[rows(r), rcols]
            )
            ccw1.append(mk_ccw(1, r, recv_ccw.at[0, rows(r)], recv_ccw.at[1, rows(r)]))
            ccw1[r].start()
        for r in range(R):
            cw0[r].wait_send()
            ccw0[r].wait_send()

        out_ref[:, lcols] = xdot(1, lcols)
        out_ref[:, rcols] = xdot(0, rcols)

        for r in range(R):
            cw1[r].wait_send()
            pl.semaphore_signal(
                credit_cw, inc=1,
                device_id=(left,), device_id_type=pl.DeviceIdType.MESH,
            )
            ccw1[r].wait_send()
            pl.semaphore_signal(
                credit_ccw, inc=1,
                device_id=(right,), device_id_type=pl.DeviceIdType.MESH,
            )

        cw2, ccw2 = [], []
        for r in range(R):
            cw1[r].wait_recv()
            recv_cw[1, rows(r), :] = (
                recv_cw[1, rows(r), :] + out_ref[rows(r), lcols]
            )
            pl.semaphore_wait(credit_cw, 1)
            cw2.append(mk_cw(2, r, recv_cw.at[1, rows(r)], recv_cw.at[0, rows(r)]))
            cw2[r].start()
            ccw1[r].wait_recv()
            recv_ccw[1, rows(r), :] = (
                recv_ccw[1, rows(r), :] + out_ref[rows(r), rcols]
            )
            pl.semaphore_wait(credit_ccw, 1)
            ccw2.append(mk_ccw(2, r, recv_ccw.at[1, rows(r)], recv_ccw.at[0, rows(r)]))
            ccw2[r].start()

        loads[3].wait()
        out_ref[:, lcols] = xdot(3, lcols)
        out_ref[:, rcols] = xdot(3, rcols)

        for r in range(R):
            cw2[r].wait_recv()
            out_ref[rows(r), lcols] = jnp.maximum(
                recv_cw[0, rows(r), :] + out_ref[rows(r), lcols], 0.0
            )
            ccw2[r].wait_recv()
            out_ref[rows(r), rcols] = jnp.maximum(
                recv_ccw[0, rows(r), :] + out_ref[rows(r), rcols], 0.0
            )
        for r in range(R):
            cw2[r].wait_send()
            ccw2[r].wait_send()

    return pl.pallas_call(
        body,
        out_shape=jax.ShapeDtypeStruct((m_per, n), jnp.float32),
        in_specs=[
            pl.BlockSpec(memory_space=pltpu.MemorySpace.HBM),
            pl.BlockSpec(memory_space=pltpu.MemorySpace.HBM),
        ],
        out_specs=pl.BlockSpec(memory_space=pltpu.VMEM),
        scratch_shapes=[
            pltpu.VMEM((2, m_per, half), jnp.float32),
            pltpu.VMEM((2, m_per, half), jnp.float32),
            pltpu.VMEM((2, m_per, half), jnp.float32),
            pltpu.VMEM((N_DEV, m_per, k_per), jnp.float32),
            pltpu.VMEM((k_per, n), jnp.float32),
            pltpu.SemaphoreType.DMA(((N_DEV - 1) * R,)),
            pltpu.SemaphoreType.DMA(((N_DEV - 1) * R,)),
            pltpu.SemaphoreType.DMA(((N_DEV - 1) * R,)),
            pltpu.SemaphoreType.DMA(((N_DEV - 1) * R,)),
            pltpu.SemaphoreType.DMA((N_DEV + 1,)),
            pltpu.SemaphoreType.REGULAR,
            pltpu.SemaphoreType.REGULAR,
        ],
        compiler_params=pltpu.CompilerParams(
            collective_id=0,
            vmem_limit_bytes=128 * 1024 * 1024,
        ),
    )(x, w_mat)


# device time: 154800 ns/iter; 1.1694x vs baseline; 1.0118x over previous
import jax
import jax.numpy as jnp
from jax import lax
from jax.experimental import pallas as pl
from jax.experimental.pallas import tpu as pltpu

N_DEV = 4
R = 4


def kernel(x, w_mat):
    m_total, k_per = x.shape
    k_per2, n = w_mat.shape
    assert k_per == k_per2
    m_per = m_total // N_DEV
    half = n // 2
    sub = m_per // R

    def body(x_hbm, w_hbm, out_ref,
             p0, recv_cw, recv_ccw, xc, wv,
             cw_send_sems, cw_recv_sems, ccw_send_sems, ccw_recv_sems,
             load_sems, credit_cw, credit_ccw):
        d = lax.axis_index("i")
        left = lax.rem(d + N_DEV - 1, N_DEV)
        right = lax.rem(d + 1, N_DEV)

        lcols = pl.ds(0, half)
        rcols = pl.ds(half, half)

        def rows(r):
            return pl.ds(r * sub, sub)

        XOFF = (N_DEV - 1, 1, 2, 0)
        coffs = [lax.rem(d + off, N_DEV) * m_per for off in XOFF]
        subloads = {}
        for j in (0, 1):
            for r in range(R):
                subloads[j, r] = pltpu.make_async_copy(
                    x_hbm.at[pl.ds(coffs[j] + r * sub, sub), :],
                    xc.at[j, rows(r)],
                    load_sems.at[j * R + r],
                )
        loads = {}
        for j in (2, 3):
            loads[j] = pltpu.make_async_copy(
                x_hbm.at[pl.ds(coffs[j], m_per), :], xc.at[j],
                load_sems.at[2 * R + j - 2],
            )
        wloads = [
            pltpu.make_async_copy(
                w_hbm.at[:, pl.ds(h * half, half)], wv.at[:, pl.ds(h * half, half)],
                load_sems.at[2 * R + 2 + h],
            )
            for h in (0, 1)
        ]
        subloads[0, 0].start()
        wloads[0].start()
        subloads[1, 0].start()
        wloads[1].start()
        for r in range(1, R):
            subloads[0, r].start()
            subloads[1, r].start()
        loads[2].start()
        loads[3].start()

        def xdot(j, w_cols, r=None):
            off = 0 if r is None else r * sub
            return jnp.dot(
                xc[j, pl.ds(off, m_per if r is None else sub), :],
                wv[:, w_cols],
                preferred_element_type=jnp.float32,
            )

        def mk(s, r, src, dst, send_sems, recv_sems, tgt):
            return pltpu.make_async_remote_copy(
                src_ref=src, dst_ref=dst,
                send_sem=send_sems.at[s * R + r],
                recv_sem=recv_sems.at[s * R + r],
                device_id=(tgt,), device_id_type=pl.DeviceIdType.MESH,
            )

        def mk_cw(s, r, src, dst):
            return mk(s, r, src, dst, cw_send_sems, cw_recv_sems, right)

        def mk_ccw(s, r, src, dst):
            return mk(s, r, src, dst, ccw_send_sems, ccw_recv_sems, left)

        barrier_sem = pltpu.get_barrier_semaphore()
        for nbr in (left, right):
            pl.semaphore_signal(
                barrier_sem, inc=1,
                device_id=(nbr,), device_id_type=pl.DeviceIdType.MESH,
            )
        pl.semaphore_wait(barrier_sem, 2)

        cw0, ccw0 = [], []
        for r in range(R):
            subloads[0, r].wait()
            if r == 0:
                wloads[0].wait()
            p0[0, rows(r), :] = xdot(0, lcols, r)
            cw0.append(mk_cw(0, r, p0.at[0, rows(r)], recv_cw.at[0, rows(r)]))
            cw0[r].start()
            subloads[1, r].wait()
            if r == 0:
                wloads[1].wait()
            p0[1, rows(r), :] = xdot(1, rcols, r)
            ccw0.append(mk_ccw(0, r, p0.at[1, rows(r)], recv_ccw.at[0, rows(r)]))
            ccw0[r].start()

        loads[2].wait()
        out_ref[:, lcols] = xdot(2, lcols)
        out_ref[:, rcols] = xdot(2, rcols)

        cw1, ccw1 = [], []
        for r in range(R):
            cw0[r].wait_recv()
            recv_cw[0, rows(r), :] = (
                recv_cw[0, rows(r), :] + out_ref[rows(r), lcols]
            )
            cw1.append(mk_cw(1, r, recv_cw.at[0, rows(r)], recv_cw.at[1, rows(r)]))
            cw1[r].start()
            ccw0[r].wait_recv()
            recv_ccw[0, rows(r), :] = (
                recv_ccw[0, rows(r), :] + out_ref[rows(r), rcols]
            )
            ccw1.append(mk_ccw(1, r, recv_ccw.at[0, rows(r)], recv_ccw.at[1, rows(r)]))
            ccw1[r].start()
        for r in range(R):
            cw0[r].wait_send()
            ccw0[r].wait_send()

        out_ref[:, lcols] = xdot(1, lcols)
        out_ref[:, rcols] = xdot(0, rcols)

        for r in range(R):
            cw1[r].wait_send()
            pl.semaphore_signal(
                credit_cw, inc=1,
                device_id=(left,), device_id_type=pl.DeviceIdType.MESH,
            )
            ccw1[r].wait_send()
            pl.semaphore_signal(
                credit_ccw, inc=1,
                device_id=(right,), device_id_type=pl.DeviceIdType.MESH,
            )

        cw2, ccw2 = [], []
        for r in range(R):
            cw1[r].wait_recv()
            recv_cw[1, rows(r), :] = (
                recv_cw[1, rows(r), :] + out_ref[rows(r), lcols]
            )
            pl.semaphore_wait(credit_cw, 1)
            cw2.append(mk_cw(2, r, recv_cw.at[1, rows(r)], recv_cw.at[0, rows(r)]))
            cw2[r].start()
            ccw1[r].wait_recv()
            recv_ccw[1, rows(r), :] = (
                recv_ccw[1, rows(r), :] + out_ref[rows(r), rcols]
            )
            pl.semaphore_wait(credit_ccw, 1)
            ccw2.append(mk_ccw(2, r, recv_ccw.at[1, rows(r)], recv_ccw.at[0, rows(r)]))
            ccw2[r].start()

        loads[3].wait()
        out_ref[:, lcols] = xdot(3, lcols)
        out_ref[:, rcols] = xdot(3, rcols)

        for r in range(R):
            cw2[r].wait_recv()
            out_ref[rows(r), lcols] = jnp.maximum(
                recv_cw[0, rows(r), :] + out_ref[rows(r), lcols], 0.0
            )
            ccw2[r].wait_recv()
            out_ref[rows(r), rcols] = jnp.maximum(
                recv_ccw[0, rows(r), :] + out_ref[rows(r), rcols], 0.0
            )
        for r in range(R):
            cw2[r].wait_send()
            ccw2[r].wait_send()

    return pl.pallas_call(
        body,
        out_shape=jax.ShapeDtypeStruct((m_per, n), jnp.float32),
        in_specs=[
            pl.BlockSpec(memory_space=pltpu.MemorySpace.HBM),
            pl.BlockSpec(memory_space=pltpu.MemorySpace.HBM),
        ],
        out_specs=pl.BlockSpec(memory_space=pltpu.VMEM),
        scratch_shapes=[
            pltpu.VMEM((2, m_per, half), jnp.float32),
            pltpu.VMEM((2, m_per, half), jnp.float32),
            pltpu.VMEM((2, m_per, half), jnp.float32),
            pltpu.VMEM((N_DEV, m_per, k_per), jnp.float32),
            pltpu.VMEM((k_per, n), jnp.float32),
            pltpu.SemaphoreType.DMA(((N_DEV - 1) * R,)),
            pltpu.SemaphoreType.DMA(((N_DEV - 1) * R,)),
            pltpu.SemaphoreType.DMA(((N_DEV - 1) * R,)),
            pltpu.SemaphoreType.DMA(((N_DEV - 1) * R,)),
            pltpu.SemaphoreType.DMA((2 * R + 4,)),
            pltpu.SemaphoreType.REGULAR,
            pltpu.SemaphoreType.REGULAR,
        ],
        compiler_params=pltpu.CompilerParams(
            collective_id=0,
            vmem_limit_bytes=128 * 1024 * 1024,
        ),
    )(x, w_mat)


# device time: 154757 ns/iter; 1.1697x vs baseline; 1.0003x over previous
import jax
import jax.numpy as jnp
from jax import lax
from jax.experimental import pallas as pl
from jax.experimental.pallas import tpu as pltpu

N_DEV = 4
R = 8


def kernel(x, w_mat):
    m_total, k_per = x.shape
    k_per2, n = w_mat.shape
    assert k_per == k_per2
    m_per = m_total // N_DEV
    half = n // 2
    sub = m_per // R

    def body(x_hbm, w_hbm, out_ref,
             p0, recv_cw, recv_ccw, xc, wv,
             cw_send_sems, cw_recv_sems, ccw_send_sems, ccw_recv_sems,
             load_sems, credit_cw, credit_ccw):
        d = lax.axis_index("i")
        left = lax.rem(d + N_DEV - 1, N_DEV)
        right = lax.rem(d + 1, N_DEV)

        lcols = pl.ds(0, half)
        rcols = pl.ds(half, half)

        def rows(r):
            return pl.ds(r * sub, sub)

        XOFF = (N_DEV - 1, 1, 2, 0)
        coffs = [lax.rem(d + off, N_DEV) * m_per for off in XOFF]
        subloads = {}
        for j in (0, 1):
            for r in range(R):
                subloads[j, r] = pltpu.make_async_copy(
                    x_hbm.at[pl.ds(coffs[j] + r * sub, sub), :],
                    xc.at[j, rows(r)],
                    load_sems.at[j * R + r],
                )
        loads = {}
        for j in (2, 3):
            loads[j] = pltpu.make_async_copy(
                x_hbm.at[pl.ds(coffs[j], m_per), :], xc.at[j],
                load_sems.at[2 * R + j - 2],
            )
        wloads = [
            pltpu.make_async_copy(
                w_hbm.at[:, pl.ds(h * half, half)], wv.at[:, pl.ds(h * half, half)],
                load_sems.at[2 * R + 2 + h],
            )
            for h in (0, 1)
        ]
        subloads[0, 0].start()
        wloads[0].start()
        subloads[1, 0].start()
        wloads[1].start()
        for r in range(1, R):
            subloads[0, r].start()
            subloads[1, r].start()
        loads[2].start()
        loads[3].start()

        def xdot(j, w_cols, r=None):
            off = 0 if r is None else r * sub
            return jnp.dot(
                xc[j, pl.ds(off, m_per if r is None else sub), :],
                wv[:, w_cols],
                preferred_element_type=jnp.float32,
            )

        def mk(s, r, src, dst, send_sems, recv_sems, tgt):
            return pltpu.make_async_remote_copy(
                src_ref=src, dst_ref=dst,
                send_sem=send_sems.at[s * R + r],
                recv_sem=recv_sems.at[s * R + r],
                device_id=(tgt,), device_id_type=pl.DeviceIdType.MESH,
            )

        def mk_cw(s, r, src, dst):
            return mk(s, r, src, dst, cw_send_sems, cw_recv_sems, right)

        def mk_ccw(s, r, src, dst):
            return mk(s, r, src, dst, ccw_send_sems, ccw_recv_sems, left)

        barrier_sem = pltpu.get_barrier_semaphore()
        for nbr in (left, right):
            pl.semaphore_signal(
                barrier_sem, inc=1,
                device_id=(nbr,), device_id_type=pl.DeviceIdType.MESH,
            )
        pl.semaphore_wait(barrier_sem, 2)

        cw0, ccw0 = [], []
        for r in range(R):
            subloads[0, r].wait()
            if r == 0:
                wloads[0].wait()
            p0[0, rows(r), :] = xdot(0, lcols, r)
            cw0.append(mk_cw(0, r, p0.at[0, rows(r)], recv_cw.at[0, rows(r)]))
            cw0[r].start()
            subloads[1, r].wait()
            if r == 0:
                wloads[1].wait()
            p0[1, rows(r), :] = xdot(1, rcols, r)
            ccw0.append(mk_ccw(0, r, p0.at[1, rows(r)], recv_ccw.at[0, rows(r)]))
            ccw0[r].start()

        loads[2].wait()
        out_ref[:, lcols] = xdot(2, lcols)
        out_ref[:, rcols] = xdot(2, rcols)

        cw1, ccw1 = [], []
        for r in range(R):
            cw0[r].wait_recv()
            recv_cw[0, rows(r), :] = (
                recv_cw[0, rows(r), :] + out_ref[rows(r), lcols]
            )
            cw1.append(mk_cw(1, r, recv_cw.at[0, rows(r)], recv_cw.at[1, rows(r)]))
            cw1[r].start()
            ccw0[r].wait_recv()
            recv_ccw[0, rows(r), :] = (
                recv_ccw[0, rows(r), :] + out_ref[rows(r), rcols]
            )
            ccw1.append(mk_ccw(1, r, recv_ccw.at[0, rows(r)], recv_ccw.at[1, rows(r)]))
            ccw1[r].start()
        for r in range(R):
            cw0[r].wait_send()
            ccw0[r].wait_send()

        out_ref[:, lcols] = xdot(1, lcols)
        out_ref[:, rcols] = xdot(0, rcols)

        for r in range(R):
            cw1[r].wait_send()
            pl.semaphore_signal(
                credit_cw, inc=1,
                device_id=(left,), device_id_type=pl.DeviceIdType.MESH,
            )
            ccw1[r].wait_send()
            pl.semaphore_signal(
                credit_ccw, inc=1,
                device_id=(right,), device_id_type=pl.DeviceIdType.MESH,
            )

        cw2, ccw2 = [], []
        for r in range(R):
            cw1[r].wait_recv()
            recv_cw[1, rows(r), :] = (
                recv_cw[1, rows(r), :] + out_ref[rows(r), lcols]
            )
            pl.semaphore_wait(credit_cw, 1)
            cw2.append(mk_cw(2, r, recv_cw.at[1, rows(r)], recv_cw.at[0, rows(r)]))
            cw2[r].start()
            ccw1[r].wait_recv()
            recv_ccw[1, rows(r), :] = (
                recv_ccw[1, rows(r), :] + out_ref[rows(r), rcols]
            )
            pl.semaphore_wait(credit_ccw, 1)
            ccw2.append(mk_ccw(2, r, recv_ccw.at[1, rows(r)], recv_ccw.at[0, rows(r)]))
            ccw2[r].start()

        loads[3].wait()
        out_ref[:, lcols] = xdot(3, lcols)
        out_ref[:, rcols] = xdot(3, rcols)

        for r in range(R):
            cw2[r].wait_recv()
            out_ref[rows(r), lcols] = jnp.maximum(
                recv_cw[0, rows(r), :] + out_ref[rows(r), lcols], 0.0
            )
            ccw2[r].wait_recv()
            out_ref[rows(r), rcols] = jnp.maximum(
                recv_ccw[0, rows(r), :] + out_ref[rows(r), rcols], 0.0
            )
        for r in range(R):
            cw2[r].wait_send()
            ccw2[r].wait_send()

    return pl.pallas_call(
        body,
        out_shape=jax.ShapeDtypeStruct((m_per, n), jnp.float32),
        in_specs=[
            pl.BlockSpec(memory_space=pltpu.MemorySpace.HBM),
            pl.BlockSpec(memory_space=pltpu.MemorySpace.HBM),
        ],
        out_specs=pl.BlockSpec(memory_space=pltpu.VMEM),
        scratch_shapes=[
            pltpu.VMEM((2, m_per, half), jnp.float32),
            pltpu.VMEM((2, m_per, half), jnp.float32),
            pltpu.VMEM((2, m_per, half), jnp.float32),
            pltpu.VMEM((N_DEV, m_per, k_per), jnp.float32),
            pltpu.VMEM((k_per, n), jnp.float32),
            pltpu.SemaphoreType.DMA(((N_DEV - 1) * R,)),
            pltpu.SemaphoreType.DMA(((N_DEV - 1) * R,)),
            pltpu.SemaphoreType.DMA(((N_DEV - 1) * R,)),
            pltpu.SemaphoreType.DMA(((N_DEV - 1) * R,)),
            pltpu.SemaphoreType.DMA((2 * R + 4,)),
            pltpu.SemaphoreType.REGULAR,
            pltpu.SemaphoreType.REGULAR,
        ],
        compiler_params=pltpu.CompilerParams(
            collective_id=0,
            vmem_limit_bytes=128 * 1024 * 1024,
        ),
    )(x, w_mat)


# device time: 153073 ns/iter; 1.1826x vs baseline; 1.0110x over previous
import jax
import jax.numpy as jnp
from jax import lax
from jax.experimental import pallas as pl
from jax.experimental.pallas import tpu as pltpu

N_DEV = 4
R = 4


def kernel(x, w_mat):
    m_total, k_per = x.shape
    k_per2, n = w_mat.shape
    assert k_per == k_per2
    m_per = m_total // N_DEV
    half = n // 2
    sub = m_per // R

    def body(x_hbm, w_hbm, out_hbm,
             p0, recv_cw, recv_ccw, xc, wv, stage,
             cw_send_sems, cw_recv_sems, ccw_send_sems, ccw_recv_sems,
             load_sems, out_sems, credit_cw, credit_ccw):
        d = lax.axis_index("i")
        left = lax.rem(d + N_DEV - 1, N_DEV)
        right = lax.rem(d + 1, N_DEV)

        lcols = pl.ds(0, half)
        rcols = pl.ds(half, half)

        def rows(r):
            return pl.ds(r * sub, sub)

        XOFF = (N_DEV - 1, 1, 2, 0)
        coffs = [lax.rem(d + off, N_DEV) * m_per for off in XOFF]
        subloads = {}
        for j in (0, 1):
            for r in range(R):
                subloads[j, r] = pltpu.make_async_copy(
                    x_hbm.at[pl.ds(coffs[j] + r * sub, sub), :],
                    xc.at[j, rows(r)],
                    load_sems.at[j * R + r],
                )
        loads = {}
        for j in (2, 3):
            loads[j] = pltpu.make_async_copy(
                x_hbm.at[pl.ds(coffs[j], m_per), :], xc.at[j],
                load_sems.at[2 * R + j - 2],
            )
        wloads = [
            pltpu.make_async_copy(
                w_hbm.at[:, pl.ds(h * half, half)], wv.at[:, pl.ds(h * half, half)],
                load_sems.at[2 * R + 2 + h],
            )
            for h in (0, 1)
        ]
        subloads[0, 0].start()
        wloads[0].start()
        subloads[1, 0].start()
        wloads[1].start()
        for r in range(1, R):
            subloads[0, r].start()
            subloads[1, r].start()
        loads[2].start()
        loads[3].start()

        def xdot(j, w_cols, r=None):
            off = 0 if r is None else r * sub
            return jnp.dot(
                xc[j, pl.ds(off, m_per if r is None else sub), :],
                wv[:, w_cols],
                preferred_element_type=jnp.float32,
            )

        def mk(s, r, src, dst, send_sems, recv_sems, tgt):
            return pltpu.make_async_remote_copy(
                src_ref=src, dst_ref=dst,
                send_sem=send_sems.at[s * R + r],
                recv_sem=recv_sems.at[s * R + r],
                device_id=(tgt,), device_id_type=pl.DeviceIdType.MESH,
            )

        def mk_cw(s, r, src, dst):
            return mk(s, r, src, dst, cw_send_sems, cw_recv_sems, right)

        def mk_ccw(s, r, src, dst):
            return mk(s, r, src, dst, ccw_send_sems, ccw_recv_sems, left)

        barrier_sem = pltpu.get_barrier_semaphore()
        for nbr in (left, right):
            pl.semaphore_signal(
                barrier_sem, inc=1,
                device_id=(nbr,), device_id_type=pl.DeviceIdType.MESH,
            )
        pl.semaphore_wait(barrier_sem, 2)

        cw0, ccw0 = [], []
        for r in range(R):
            subloads[0, r].wait()
            if r == 0:
                wloads[0].wait()
            p0[0, rows(r), :] = xdot(0, lcols, r)
            cw0.append(mk_cw(0, r, p0.at[0, rows(r)], recv_cw.at[0, rows(r)]))
            cw0[r].start()
            subloads[1, r].wait()
            if r == 0:
                wloads[1].wait()
            p0[1, rows(r), :] = xdot(1, rcols, r)
            ccw0.append(mk_ccw(0, r, p0.at[1, rows(r)], recv_ccw.at[0, rows(r)]))
            ccw0[r].start()

        loads[2].wait()
        stage[:, lcols] = xdot(2, lcols)
        stage[:, rcols] = xdot(2, rcols)

        cw1, ccw1 = [], []
        for r in range(R):
            cw0[r].wait_recv()
            recv_cw[0, rows(r), :] = (
                recv_cw[0, rows(r), :] + stage[rows(r), lcols]
            )
            cw1.append(mk_cw(1, r, recv_cw.at[0, rows(r)], recv_cw.at[1, rows(r)]))
            cw1[r].start()
            ccw0[r].wait_recv()
            recv_ccw[0, rows(r), :] = (
                recv_ccw[0, rows(r), :] + stage[rows(r), rcols]
            )
            ccw1.append(mk_ccw(1, r, recv_ccw.at[0, rows(r)], recv_ccw.at[1, rows(r)]))
            ccw1[r].start()
        for r in range(R):
            cw0[r].wait_send()
            ccw0[r].wait_send()

        stage[:, lcols] = xdot(1, lcols)
        stage[:, rcols] = xdot(0, rcols)

        for r in range(R):
            cw1[r].wait_send()
            pl.semaphore_signal(
                credit_cw, inc=1,
                device_id=(left,), device_id_type=pl.DeviceIdType.MESH,
            )
            ccw1[r].wait_send()
            pl.semaphore_signal(
                credit_ccw, inc=1,
                device_id=(right,), device_id_type=pl.DeviceIdType.MESH,
            )

        cw2, ccw2 = [], []
        for r in range(R):
            cw1[r].wait_recv()
            recv_cw[1, rows(r), :] = (
                recv_cw[1, rows(r), :] + stage[rows(r), lcols]
            )
            pl.semaphore_wait(credit_cw, 1)
            cw2.append(mk_cw(2, r, recv_cw.at[1, rows(r)], recv_cw.at[0, rows(r)]))
            cw2[r].start()
            ccw1[r].wait_recv()
            recv_ccw[1, rows(r), :] = (
                recv_ccw[1, rows(r), :] + stage[rows(r), rcols]
            )
            pl.semaphore_wait(credit_ccw, 1)
            ccw2.append(mk_ccw(2, r, recv_ccw.at[1, rows(r)], recv_ccw.at[0, rows(r)]))
            ccw2[r].start()

        loads[3].wait()
        stage[:, lcols] = xdot(3, lcols)
        stage[:, rcols] = xdot(3, rcols)

        outs = []
        for r in range(R):
            cw2[r].wait_recv()
            stage[rows(r), lcols] = jnp.maximum(
                recv_cw[0, rows(r), :] + stage[rows(r), lcols], 0.0
            )
            ccw2[r].wait_recv()
            stage[rows(r), rcols] = jnp.maximum(
                recv_ccw[0, rows(r), :] + stage[rows(r), rcols], 0.0
            )
            outs.append(pltpu.make_async_copy(
                stage.at[rows(r), :], out_hbm.at[rows(r), :], out_sems.at[r],
            ))
            outs[r].start()
        for r in range(R):
            outs[r].wait()
            cw2[r].wait_send()
            ccw2[r].wait_send()

    return pl.pallas_call(
        body,
        out_shape=jax.ShapeDtypeStruct((m_per, n), jnp.float32),
        in_specs=[
            pl.BlockSpec(memory_space=pltpu.MemorySpace.HBM),
            pl.BlockSpec(memory_space=pltpu.MemorySpace.HBM),
        ],
        out_specs=pl.BlockSpec(memory_space=pltpu.MemorySpace.HBM),
        scratch_shapes=[
            pltpu.VMEM((2, m_per, half), jnp.float32),
            pltpu.VMEM((2, m_per, half), jnp.float32),
            pltpu.VMEM((2, m_per, half), jnp.float32),
            pltpu.VMEM((N_DEV, m_per, k_per), jnp.float32),
            pltpu.VMEM((k_per, n), jnp.float32),
            pltpu.VMEM((m_per, n), jnp.float32),
            pltpu.SemaphoreType.DMA(((N_DEV - 1) * R,)),
            pltpu.SemaphoreType.DMA(((N_DEV - 1) * R,)),
            pltpu.SemaphoreType.DMA(((N_DEV - 1) * R,)),
            pltpu.SemaphoreType.DMA(((N_DEV - 1) * R,)),
            pltpu.SemaphoreType.DMA((2 * R + 4,)),
            pltpu.SemaphoreType.DMA((R,)),
            pltpu.SemaphoreType.REGULAR,
            pltpu.SemaphoreType.REGULAR,
        ],
        compiler_params=pltpu.CompilerParams(
            collective_id=0,
            vmem_limit_bytes=128 * 1024 * 1024,
        ),
    )(x, w_mat)
